# baseline (speedup 1.0000x reference)
import jax
import jax.numpy as jnp
from jax.experimental import pallas as pl

N = 10000
E = 320000
G = 64
H = 256
OUT_CH = 32
NUM_LAYERS = 4
NUM_TIMESTEPS = 4


def _gru(p, x, h):
    wi, wh, bi, bh = p
    gi = x @ wi.T + bi
    gh = h @ wh.T + bh
    i_r, i_z, i_n = jnp.split(gi, 3, axis=-1)
    h_r, h_z, h_n = jnp.split(gh, 3, axis=-1)
    r = jax.nn.sigmoid(i_r + h_r)
    z = jax.nn.sigmoid(i_z + h_z)
    n = jnp.tanh(i_n + r * h_n)
    return (1.0 - z) * n + z * h


def _segment_softmax(alpha, index, num_segments):
    amax = jax.ops.segment_max(alpha, index, num_segments=num_segments)
    amax = jnp.where(jnp.isfinite(amax), amax, 0.0)
    ex = jnp.exp(alpha - amax[index])
    s = jax.ops.segment_sum(ex, index, num_segments=num_segments)
    return ex / (s[index] + 1e-16)


def _final_proj_kernel(x_ref, w_ref, b_ref, o_ref):
    o_ref[...] = x_ref[...] @ w_ref[...] + b_ref[...]


def kernel(x, edge_index, edge_attr, batch, params):
    p = params
    src = edge_index[0]
    dst = edge_index[1]
    x = jax.nn.leaky_relu(x @ p['lin1_w'] + p['lin1_b'], 0.01)
    xj = x[src]
    xi = x[dst]
    t = jax.nn.leaky_relu(
        jnp.concatenate([xj, edge_attr], axis=-1) @ p['gate_lin1_w'], 0.01)
    alpha = jax.nn.leaky_relu(t @ p['gate_att_l'] + xi @ p['gate_att_r'], 0.01)
    alpha = _segment_softmax(alpha, dst, N)
    msg = (xj @ p['gate_lin2_w']) * alpha[:, None]
    h = jax.ops.segment_sum(msg, dst, num_segments=N) + p['gate_bias']
    h = jax.nn.elu(h)
    x = jax.nn.relu(_gru(p['gru0'], h, x))
    for conv, gru in zip(p['atom_convs'], p['atom_grus']):
        xs = x @ conv['lin_w']
        a_src = xs @ conv['att_src']
        a_dst = xs @ conv['att_dst']
        alpha = jax.nn.leaky_relu(a_src[src] + a_dst[dst], 0.01)
        alpha = _segment_softmax(alpha, dst, N)
        h = jax.ops.segment_sum(xs[src] * alpha[:, None], dst,
                                num_segments=N) + conv['bias']
        h = jax.nn.elu(h)
        x = jax.nn.relu(_gru(gru, h, x))
    out = jax.nn.relu(jax.ops.segment_sum(x, batch, num_segments=G))
    mc = p['mol_conv']
    xs = x @ mc['lin_w']
    a_src_all = xs @ mc['att_src']
    for _ in range(NUM_TIMESTEPS):
        od = out @ mc['lin_w']
        a_dst = od @ mc['att_dst']
        alpha = jax.nn.leaky_relu(a_src_all + a_dst[batch], 0.01)
        alpha = _segment_softmax(alpha, batch, G)
        h = jax.ops.segment_sum(xs * alpha[:, None], batch,
                                num_segments=G) + mc['bias']
        h = jax.nn.elu(h)
        out = jax.nn.relu(_gru(p['mol_gru'], h, out))
    return pl.pallas_call(
        _final_proj_kernel,
        out_shape=jax.ShapeDtypeStruct((G, OUT_CH), jnp.float32),
    )(out, p['lin2_w'], jnp.broadcast_to(p['lin2_b'], (G, OUT_CH)))


# speedup vs baseline: 3.1697x; 3.1697x over previous
import functools

import jax
import jax.numpy as jnp
from jax import lax
from jax.experimental import pallas as pl
from jax.experimental.pallas import tpu as pltpu
from jax.experimental.pallas import tpu_sc as plsc

N = 10000
E = 320000
G = 64
H = 256
HH = H // 2
OUT_CH = 32
NUM_TIMESTEPS = 4

NHALF = N // 2
NTEC = 16
ETP = 20480
EP = ETP * NTEC
QTR = ETP // 4
RGA = 32
RGB = 16
DUMP = 5119
SROWS = 5120
NEG = -3.0e38


def _lrelu(v):
    return jnp.where(v >= 0, v, 0.01 * v)


def _elu(v):
    return jnp.where(v > 0, v, jnp.exp(jnp.minimum(v, 0.0)) - 1.0)


def _vperm(v, idx):
    return lax.gather(
        v, idx[:, None],
        dimension_numbers=lax.GatherDimensionNumbers(
            offset_dims=(), collapsed_slice_dims=(0,), start_index_map=(0,)),
        slice_sizes=(1,),
        mode=lax.GatherScatterMode.PROMISE_IN_BOUNDS)


def _splat_max(v):
    iota16 = lax.iota(jnp.int32, 16)
    for step in (8, 4, 2, 1):
        v = jnp.maximum(v, _vperm(v, iota16 ^ step))
    return v


def _splat_sum(v):
    iota16 = lax.iota(jnp.int32, 16)
    for step in (8, 4, 2, 1):
        v = v + _vperm(v, iota16 ^ step)
    return v


def _prep_body(x_ref, l1w, l1b, w1a, w2, attr_ref, x1o, g1o, g2o, aro):
    x1 = _lrelu(x_ref[...] @ l1w[...] + l1b[...])
    x1o[...] = x1
    g1o[...] = x1 @ w1a[...]
    g2o[...] = x1 @ w2[...]
    aro[...] = x1 @ attr_ref[...][:, None]


def _gru_math(gi, gh, x_old):
    r = jax.nn.sigmoid(gi[:, :H] + gh[:, :H])
    z = jax.nn.sigmoid(gi[:, H:2 * H] + gh[:, H:2 * H])
    n = jnp.tanh(gi[:, 2 * H:] + r * gh[:, 2 * H:])
    return jnp.maximum((1.0 - z) * n + z * x_old, 0.0)


def _post_body(hraw, xold, bias, wit, wht, bi, bh, nlw, nas, nad,
               xnew, xso, aso, ado):
    h = _elu(hraw[...] + bias[...])
    gi = h @ wit[...] + bi[...]
    gh = xold[...] @ wht[...] + bh[...]
    xn = _gru_math(gi, gh, xold[...])
    xnew[...] = xn
    xs = xn @ nlw[...]
    xso[...] = xs
    aso[...] = xs @ nas[...][:, None]
    ado[...] = xs @ nad[...][:, None]


def _mol_body(x_ref, xs_ref, asrc_ref, batch_ref, mlw, mad, mbias,
              wit, wht, bi, bh, l2w, l2b, outo):
    batch = batch_ref[...]
    oh_b = batch[:, None] == lax.broadcasted_iota(jnp.int32, (N, G), 1)
    oh = oh_b.astype(jnp.float32)
    x = x_ref[...]
    xs = xs_ref[...]
    a_src = asrc_ref[...]
    dn = (((0,), (0,)), ((), ()))
    out = jnp.maximum(lax.dot_general(oh, x, dn), 0.0)
    for _ in range(NUM_TIMESTEPS):
        od = out @ mlw[...]
        a_dst = od @ mad[...][:, None]
        alpha = _lrelu(a_src + (oh @ a_dst)[:, 0])
        ex = jnp.exp(alpha - jnp.max(alpha))
        s = lax.dot_general(oh, ex[:, None], dn)
        w = ex / ((oh @ s)[:, 0] + 1e-16)
        hm = _elu(lax.dot_general(oh * w[:, None], xs, dn) + mbias[...])
        gi = hm @ wit[...] + bi[...]
        gh = out @ wht[...] + bh[...]
        out = _gru_math(gi, gh, out)
    outo[...] = out @ l2w[...] + l2b[...]


def _stage_idx(stg, src_ref, off, n, add=0):
    for q in range(n // 16):
        v = src_ref[pl.ds(off + q * 16, 16)]
        if add:
            v = v + add
        stg[0, pl.ds(q * 16, 16)] = v


def _stage_idx_clamped(stg, src_ref, off, n, hi):
    for q in range(n // 16):
        v = jnp.minimum(src_ref[pl.ds(off + q * 16, 16)], hi)
        stg[0, pl.ds(q * 16, 16)] = v


def _sc_max_combine(lm, maxv, max_all, max_sh):
    s = lax.axis_index("s")
    maxv[...] = lm
    pltpu.sync_copy(maxv, max_sh.at[s])
    plsc.subcore_barrier()
    pltpu.sync_copy(max_sh, max_all)
    mv = jnp.full((16,), NEG, jnp.float32)
    for q in range(NTEC):
        mv = jnp.maximum(mv, max_all[q])
    return _splat_max(mv)


def _sc_denom_and_norm(ew, dlb, svq, stg, s_sh, zeros1_h, msplat):
    s = lax.axis_index("s")

    @pl.when(s == 0)
    def _():
        pltpu.sync_copy(zeros1_h, s_sh)

    plsc.subcore_barrier()

    def epass(g, _):
        ew[pl.ds(g * 16, 16)] = jnp.exp(ew[pl.ds(g * 16, 16)] - msplat)
        return 0

    lax.fori_loop(0, ETP // 16, epass, 0)

    def sadd(k, _):
        _stage_idx(stg, dlb, k * 128, 128)
        pltpu.sync_copy(ew.at[pl.ds(k * 128, 128)], s_sh.at[stg.at[0]],
                        add=True)
        return 0

    lax.fori_loop(0, ETP // 128, sadd, 0)
    plsc.subcore_barrier()

    for qt in range(4):
        off = qt * QTR

        def sget(k, _):
            _stage_idx(stg, dlb, off + k * 128, 128)
            pltpu.sync_copy(s_sh.at[stg.at[0]], svq.at[pl.ds(k * 128, 128)])
            return 0

        lax.fori_loop(0, QTR // 128, sget, 0)

        def wpass(g, _):
            e = ew[pl.ds(off + g * 16, 16)]
            sv = svq[pl.ds(g * 16, 16)]
            ew[pl.ds(off + g * 16, 16)] = e / (sv + 1e-16)
            return 0

        lax.fori_loop(0, QTR // 16, wpass, 0)


def _sc_aggregate(srcf, dlb, ew, rows, stg2, xs2_h, zerosc_h, h_sh, h_out,
                  sem, base, rg):
    s = lax.axis_index("s")

    for colhalf in range(2):
        @pl.when(s == 0)
        def _():
            pltpu.sync_copy(zerosc_h, h_sh)

        plsc.subcore_barrier()

        def agg(g, _):
            _stage_idx(stg2, srcf, g * rg, rg, add=colhalf * N)
            pltpu.async_copy(xs2_h.at[stg2.at[0]], rows, sem).wait()
            wgs = [ew[pl.ds(g * rg + q * 16, 16)] for q in range(rg // 16)]
            for j in range(rg):
                wj = _vperm(wgs[j // 16], jnp.full((16,), j % 16, jnp.int32))
                for ch in range(HH // 16):
                    rows[j, pl.ds(ch * 16, 16)] = (
                        rows[j, pl.ds(ch * 16, 16)] * wj)
            _stage_idx(stg2, dlb, g * rg, rg)
            pltpu.sync_copy(rows, h_sh.at[stg2.at[0]], add=True)
            return 0

        lax.fori_loop(0, ETP // rg, agg, 0)
        plsc.subcore_barrier()

        @pl.when(s == 0)
        def _():
            pltpu.sync_copy(h_sh.at[pl.ds(0, NHALF)],
                            h_out.at[pl.ds(colhalf * N + base, NHALF)])

        plsc.subcore_barrier()


def _sc_gat_body(asrc_h, adst_h, src_h, dst_h, xs2_h, zerosc_h, zeros1_h,
                 h_out,
                 srcf, dlb, ew, dsub, asgq, adg, rows, maxv, max_all,
                 stg, stg2, h_sh, s_sh, max_sh, a1_sh, a2_sh, sem):
    c = lax.axis_index("c")
    s = lax.axis_index("s")
    base = c * NHALF
    e0 = s * ETP

    @pl.when(s == 1)
    def _():
        pltpu.sync_copy(asrc_h, a1_sh)

    @pl.when(s == 2)
    def _():
        pltpu.sync_copy(adst_h, a2_sh)

    pltpu.sync_copy(src_h.at[pl.ds(e0, ETP)], srcf)
    plsc.subcore_barrier()

    lm = jnp.full((16,), NEG, jnp.float32)
    for qt in range(4):
        off = qt * QTR
        pltpu.sync_copy(dst_h.at[pl.ds(e0 + off, QTR)], dsub)

        def dpass(g, _):
            dv = dsub[pl.ds(g * 16, 16)]
            dl = dv - base
            m = (dl >= 0) & (dl < NHALF)
            dlb[pl.ds(off + g * 16, 16)] = jnp.where(m, dl, DUMP)
            return 0

        lax.fori_loop(0, QTR // 16, dpass, 0)

        def gpass_d(k, _):
            _stage_idx_clamped(stg, dsub, k * 128, 128, N - 1)
            pltpu.sync_copy(a2_sh.at[stg.at[0]], adg.at[pl.ds(k * 128, 128)])
            return 0

        lax.fori_loop(0, QTR // 128, gpass_d, 0)

        def gpass_s(k, _):
            _stage_idx(stg, srcf, off + k * 128, 128)
            pltpu.sync_copy(a1_sh.at[stg.at[0]], asgq.at[pl.ds(k * 128, 128)])
            return 0

        lax.fori_loop(0, QTR // 128, gpass_s, 0)

        def apass(g, lmc):
            av = asgq[pl.ds(g * 16, 16)]
            al = _lrelu(av + adg[pl.ds(g * 16, 16)])
            m = dlb[pl.ds(off + g * 16, 16)] < NHALF
            al = jnp.where(m, al, NEG)
            ew[pl.ds(off + g * 16, 16)] = al
            return jnp.maximum(lmc, al)

        lm = lax.fori_loop(0, QTR // 16, apass, lm)

    msplat = _sc_max_combine(lm, maxv, max_all, max_sh)
    _sc_denom_and_norm(ew, dlb, adg, stg, s_sh, zeros1_h, msplat)
    _sc_aggregate(srcf, dlb, ew, rows, stg2, xs2_h, zerosc_h, h_sh, h_out,
                  sem, base, RGA)


def _sc_gate_body(ar_h, g12_h, g22_h, wrow_h, attl_h, ea_h, src_h, dst_h,
                  zerosc_h, zeros1_h, h_out,
                  srcf, dlb, ew, dsub, svq, eab, rowsb, rows, wrow_v,
                  attl_v, maxv, max_all, stg, stg2, h_sh, s_sh, max_sh,
                  a1_sh, sem):
    c = lax.axis_index("c")
    s = lax.axis_index("s")
    base = c * NHALF
    e0 = s * ETP
    iota16 = lax.iota(jnp.int32, 16)

    @pl.when(s == 1)
    def _():
        pltpu.sync_copy(ar_h, a1_sh)

    pltpu.sync_copy(src_h.at[pl.ds(e0, ETP)], srcf)
    pltpu.sync_copy(wrow_h, wrow_v)
    pltpu.sync_copy(attl_h, attl_v)
    plsc.subcore_barrier()

    for qt in range(4):
        off = qt * QTR
        pltpu.sync_copy(dst_h.at[pl.ds(e0 + off, QTR)], dsub)

        def gpass(k, _):
            _stage_idx_clamped(stg, dsub, k * 128, 128, N - 1)
            pltpu.sync_copy(a1_sh.at[stg.at[0]],
                            svq.at[pl.ds(k * 128, 128)])
            return 0

        lax.fori_loop(0, QTR // 128, gpass, 0)

        def dpass(g, _):
            dv = dsub[pl.ds(g * 16, 16)]
            dl = dv - base
            m = (dl >= 0) & (dl < NHALF)
            dlb[pl.ds(off + g * 16, 16)] = jnp.where(m, dl, DUMP)
            arv = svq[pl.ds(g * 16, 16)]
            ew[pl.ds(off + g * 16, 16)] = jnp.where(m, arv, NEG)
            return 0

        lax.fori_loop(0, QTR // 16, dpass, 0)

    wr = [wrow_v[pl.ds(ch * 16, 16)] for ch in range(H // 16)]
    at = [attl_v[pl.ds(ch * 16, 16)] for ch in range(H // 16)]

    def bchunk(k, lmc):
        pltpu.sync_copy(ea_h.at[pl.ds(e0 + k * 128, 128)], eab)

        def bsub(u, lmc2):
            gbase = k * 128 + u * RGB
            _stage_idx(stg2, srcf, gbase, RGB, add=0)
            pltpu.async_copy(g12_h.at[stg2.at[0]], rowsb, sem).wait()
            eav = eab[pl.ds(u * RGB, 16)]
            esc = jnp.zeros((16,), jnp.float32)
            for j in range(16):
                ea_j = _vperm(eav, jnp.full((16,), j, jnp.int32))
                acc = jnp.zeros((16,), jnp.float32)
                for ch in range(H // 16):
                    y = rowsb[j, pl.ds(ch * 16, 16)] + ea_j * wr[ch]
                    y = jnp.where(y >= 0, y, 0.01 * y)
                    acc = acc + y * at[ch]
                tl = _splat_sum(acc)
                esc = jnp.where(iota16 == j, tl, esc)
            arv = ew[pl.ds(gbase, 16)]
            al = _lrelu(esc + arv)
            ew[pl.ds(gbase, 16)] = al
            return jnp.maximum(lmc2, al)

        return lax.fori_loop(0, 128 // RGB, bsub, lmc)

    lm = lax.fori_loop(0, ETP // 128, bchunk,
                       jnp.full((16,), NEG, jnp.float32))

    msplat = _sc_max_combine(lm, maxv, max_all, max_sh)
    _sc_denom_and_norm(ew, dlb, svq, stg, s_sh, zeros1_h, msplat)
    _sc_aggregate(srcf, dlb, ew, rows, stg2, g22_h, zerosc_h, h_sh, h_out,
                  sem, base, RGB)


@functools.cache
def _sc_mesh():
    return plsc.VectorSubcoreMesh(core_axis_name="c", subcore_axis_name="s")


_F32 = jnp.float32
_I32 = jnp.int32


def _sc_gat_call(asrc, adst, src, dst, xs2, zerosc, zeros1):
    return pl.kernel(
        _sc_gat_body,
        out_type=jax.ShapeDtypeStruct((2 * N, HH), _F32),
        mesh=_sc_mesh(),
        scratch_types=[
            pltpu.VMEM((ETP,), _I32),
            pltpu.VMEM((ETP,), _I32),
            pltpu.VMEM((ETP,), _F32),
            pltpu.VMEM((QTR,), _I32),
            pltpu.VMEM((QTR,), _F32),
            pltpu.VMEM((QTR,), _F32),
            pltpu.VMEM((RGA, HH), _F32),
            pltpu.VMEM((16,), _F32),
            pltpu.VMEM((16, 16), _F32),
            pltpu.VMEM((1, 128), _I32),
            pltpu.VMEM((1, RGA), _I32),
            pltpu.VMEM_SHARED((SROWS, HH), _F32),
            pltpu.VMEM_SHARED((SROWS,), _F32),
            pltpu.VMEM_SHARED((16, 16), _F32),
            pltpu.VMEM_SHARED((N,), _F32),
            pltpu.VMEM_SHARED((N,), _F32),
            pltpu.SemaphoreType.DMA,
        ],
    )(asrc, adst, src, dst, xs2, zerosc, zeros1)


def _sc_gate_call(ar, g1, g22, wrow, attl, ea, src, dst, zerosc, zeros1):
    return pl.kernel(
        _sc_gate_body,
        out_type=jax.ShapeDtypeStruct((2 * N, HH), _F32),
        mesh=_sc_mesh(),
        scratch_types=[
            pltpu.VMEM((ETP,), _I32),
            pltpu.VMEM((ETP,), _I32),
            pltpu.VMEM((ETP,), _F32),
            pltpu.VMEM((QTR,), _I32),
            pltpu.VMEM((QTR,), _F32),
            pltpu.VMEM((128,), _F32),
            pltpu.VMEM((RGB, H), _F32),
            pltpu.VMEM((RGB, HH), _F32),
            pltpu.VMEM((H,), _F32),
            pltpu.VMEM((H,), _F32),
            pltpu.VMEM((16,), _F32),
            pltpu.VMEM((16, 16), _F32),
            pltpu.VMEM((1, 128), _I32),
            pltpu.VMEM((1, RGB), _I32),
            pltpu.VMEM_SHARED((SROWS, HH), _F32),
            pltpu.VMEM_SHARED((SROWS,), _F32),
            pltpu.VMEM_SHARED((16, 16), _F32),
            pltpu.VMEM_SHARED((N,), _F32),
            pltpu.SemaphoreType.DMA,
        ],
    )(ar, g1, g22, wrow, attl, ea, src, dst, zerosc, zeros1)


_ROWS = 1000
_GRID = N // _ROWS


def _rowspec(width):
    if width is None:
        return pl.BlockSpec((_ROWS, 1), lambda i: (i, 0))
    return pl.BlockSpec((_ROWS, width), lambda i: (i, 0))


def _fullspec(shape):
    nd = len(shape)
    return pl.BlockSpec(shape, lambda i, _n=nd: (0,) * _n)


def _prep_call(x_in, p):
    return pl.pallas_call(
        _prep_body,
        grid=(_GRID,),
        in_specs=[
            _rowspec(3),
            _fullspec((3, H)), _fullspec((H,)),
            _fullspec((H, H)), _fullspec((H, H)), _fullspec((H,)),
        ],
        out_specs=[_rowspec(H), _rowspec(H), _rowspec(H), _rowspec(None)],
        out_shape=[
            jax.ShapeDtypeStruct((N, H), _F32),
            jax.ShapeDtypeStruct((N, H), _F32),
            jax.ShapeDtypeStruct((N, H), _F32),
            jax.ShapeDtypeStruct((N, 1), _F32),
        ],
    )(x_in, p['lin1_w'], p['lin1_b'], p['gate_lin1_w'][:H],
      p['gate_lin2_w'], p['gate_att_r'])


def _post_call(h_raw, x_old, bias, gru, next_w, next_as, next_ad):
    wi, wh, bi, bh = gru
    return pl.pallas_call(
        _post_body,
        grid=(_GRID,),
        in_specs=[
            _rowspec(H), _rowspec(H),
            _fullspec((H,)),
            _fullspec((H, 3 * H)), _fullspec((H, 3 * H)),
            _fullspec((3 * H,)), _fullspec((3 * H,)),
            _fullspec((H, H)), _fullspec((H,)), _fullspec((H,)),
        ],
        out_specs=[_rowspec(H), _rowspec(H), _rowspec(None),
                   _rowspec(None)],
        out_shape=[
            jax.ShapeDtypeStruct((N, H), _F32),
            jax.ShapeDtypeStruct((N, H), _F32),
            jax.ShapeDtypeStruct((N, 1), _F32),
            jax.ShapeDtypeStruct((N, 1), _F32),
        ],
    )(h_raw, x_old, bias, wi.T, wh.T, bi, bh, next_w, next_as, next_ad)


def _mol_call(x, xs, a_src, batch, mc, gru, l2w, l2b):
    wi, wh, bi, bh = gru
    return pl.pallas_call(
        _mol_body,
        in_specs=[
            pl.BlockSpec((N, H), lambda: (0, 0)),
            pl.BlockSpec((N, H), lambda: (0, 0)),
            pl.BlockSpec((N,), lambda: (0,)),
            pl.BlockSpec((N,), lambda: (0,)),
            pl.BlockSpec((H, H), lambda: (0, 0)),
            pl.BlockSpec((H,), lambda: (0,)),
            pl.BlockSpec((H,), lambda: (0,)),
            pl.BlockSpec((H, 3 * H), lambda: (0, 0)),
            pl.BlockSpec((H, 3 * H), lambda: (0, 0)),
            pl.BlockSpec((3 * H,), lambda: (0,)),
            pl.BlockSpec((3 * H,), lambda: (0,)),
            pl.BlockSpec((H, OUT_CH), lambda: (0, 0)),
            pl.BlockSpec((OUT_CH,), lambda: (0,)),
        ],
        out_specs=pl.BlockSpec((G, OUT_CH), lambda: (0, 0)),
        out_shape=jax.ShapeDtypeStruct((G, OUT_CH), _F32),
    )(x, xs, a_src, batch, mc['lin_w'], mc['att_dst'], mc['bias'],
      wi.T, wh.T, bi, bh, l2w, l2b)


def _colsplit(m):
    return m.reshape(N, 2, HH).transpose(1, 0, 2).reshape(2 * N, HH)


def _colmerge(m2):
    return m2.reshape(2, N, HH).transpose(1, 0, 2).reshape(N, H)


def kernel(x, edge_index, edge_attr, batch, params):
    p = params
    pad = EP - E
    src = jnp.concatenate([edge_index[0], jnp.zeros((pad,), _I32)])
    dst = jnp.concatenate([edge_index[1],
                           jnp.full((pad,), jnp.int32(1 << 20))])
    ea = jnp.concatenate([edge_attr[:, 0], jnp.zeros((pad,), _F32)])
    zerosc = jnp.zeros((SROWS, HH), _F32)
    zeros1 = jnp.zeros((SROWS,), _F32)

    x1, g1, g2, ar = _prep_call(x, p)
    ar = ar[:, 0]

    h2 = _sc_gate_call(ar, g1, _colsplit(g2), p['gate_lin1_w'][H],
                       p['gate_att_l'], ea, src, dst, zerosc, zeros1)
    conv0 = p['atom_convs'][0]
    xcur, xs, a_s, a_d = _post_call(
        _colmerge(h2), x1, p['gate_bias'], p['gru0'],
        conv0['lin_w'], conv0['att_src'], conv0['att_dst'])

    for li in range(3):
        conv = p['atom_convs'][li]
        gru = p['atom_grus'][li]
        h2 = _sc_gat_call(a_s[:, 0], a_d[:, 0], src, dst, _colsplit(xs),
                          zerosc, zeros1)
        if li < 2:
            nxt = p['atom_convs'][li + 1]
            nw, nas, nad = nxt['lin_w'], nxt['att_src'], nxt['att_dst']
        else:
            mc = p['mol_conv']
            nw, nas, nad = mc['lin_w'], mc['att_src'], mc['att_dst']
        xcur, xs, a_s, a_d = _post_call(
            _colmerge(h2), xcur, conv['bias'], gru, nw, nas, nad)

    return _mol_call(xcur, xs, a_s[:, 0], batch, p['mol_conv'],
                     p['mol_gru'], p['lin2_w'], p['lin2_b'])


# speedup vs baseline: 4.1179x; 1.2992x over previous
import functools

import jax
import jax.numpy as jnp
from jax import lax
from jax.experimental import pallas as pl
from jax.experimental.pallas import tpu as pltpu
from jax.experimental.pallas import tpu_sc as plsc

N = 10000
E = 320000
G = 64
H = 256
HH = H // 2
OUT_CH = 32
NUM_TIMESTEPS = 4

NHALF = N // 2
NTEC = 16
ETP = 20480
EP = ETP * NTEC
QTR = ETP // 4
RGA = 32
RGB = 16
DUMP = 5119
SROWS = 5120
NEG = -3.0e38


def _lrelu(v):
    return jnp.where(v >= 0, v, 0.01 * v)


def _elu(v):
    return jnp.where(v > 0, v, jnp.exp(jnp.minimum(v, 0.0)) - 1.0)


def _vperm(v, idx):
    return lax.gather(
        v, idx[:, None],
        dimension_numbers=lax.GatherDimensionNumbers(
            offset_dims=(), collapsed_slice_dims=(0,), start_index_map=(0,)),
        slice_sizes=(1,),
        mode=lax.GatherScatterMode.PROMISE_IN_BOUNDS)


def _splat_max(v):
    iota16 = lax.iota(jnp.int32, 16)
    for step in (8, 4, 2, 1):
        v = jnp.maximum(v, _vperm(v, iota16 ^ step))
    return v


def _splat_sum(v):
    iota16 = lax.iota(jnp.int32, 16)
    for step in (8, 4, 2, 1):
        v = v + _vperm(v, iota16 ^ step)
    return v


def _prep_body(x_ref, l1w, l1b, w1a, w2, attr_ref, x1o, g1o, g2o, aro):
    x1 = _lrelu(x_ref[...] @ l1w[...] + l1b[...])
    x1o[...] = x1
    g1o[...] = x1 @ w1a[...]
    g2o[...] = x1 @ w2[...]
    aro[...] = x1 @ attr_ref[...][:, None]


def _gru_math(gi, gh, x_old):
    r = jax.nn.sigmoid(gi[:, :H] + gh[:, :H])
    z = jax.nn.sigmoid(gi[:, H:2 * H] + gh[:, H:2 * H])
    n = jnp.tanh(gi[:, 2 * H:] + r * gh[:, 2 * H:])
    return jnp.maximum((1.0 - z) * n + z * x_old, 0.0)


def _post_body(hraw, xold, bias, wit, wht, bi, bh, nlw, nas, nad,
               xnew, xso, aso, ado):
    h = _elu(hraw[...] + bias[...])
    gi = h @ wit[...] + bi[...]
    gh = xold[...] @ wht[...] + bh[...]
    xn = _gru_math(gi, gh, xold[...])
    xnew[...] = xn
    xs = xn @ nlw[...]
    xso[...] = xs
    aso[...] = xs @ nas[...][:, None]
    ado[...] = xs @ nad[...][:, None]


def _mol_body(x_ref, xs_ref, asrc_ref, batch_ref, mlw, mad, mbias,
              wit, wht, bi, bh, l2w, l2b, outo):
    batch = batch_ref[...]
    oh_b = batch[:, None] == lax.broadcasted_iota(jnp.int32, (N, G), 1)
    oh = oh_b.astype(jnp.float32)
    x = x_ref[...]
    xs = xs_ref[...]
    a_src = asrc_ref[...]
    dn = (((0,), (0,)), ((), ()))
    out = jnp.maximum(lax.dot_general(oh, x, dn), 0.0)
    for _ in range(NUM_TIMESTEPS):
        od = out @ mlw[...]
        a_dst = od @ mad[...][:, None]
        alpha = _lrelu(a_src + (oh @ a_dst)[:, 0])
        ex = jnp.exp(alpha - jnp.max(alpha))
        s = lax.dot_general(oh, ex[:, None], dn)
        w = ex / ((oh @ s)[:, 0] + 1e-16)
        hm = _elu(lax.dot_general(oh * w[:, None], xs, dn) + mbias[...])
        gi = hm @ wit[...] + bi[...]
        gh = out @ wht[...] + bh[...]
        out = _gru_math(gi, gh, out)
    outo[...] = out @ l2w[...] + l2b[...]


def _stage_idx(stg, src_ref, off, n, add=0):
    for q in range(n // 16):
        v = src_ref[pl.ds(off + q * 16, 16)]
        if add:
            v = v + add
        stg[0, pl.ds(q * 16, 16)] = v


def _stage_idx_clamped(stg, src_ref, off, n, hi):
    for q in range(n // 16):
        v = jnp.minimum(src_ref[pl.ds(off + q * 16, 16)], hi)
        stg[0, pl.ds(q * 16, 16)] = v


def _sc_max_combine(lm, maxv, max_all, max_sh):
    s = lax.axis_index("s")
    maxv[...] = lm
    pltpu.sync_copy(maxv, max_sh.at[s])
    plsc.subcore_barrier()
    pltpu.sync_copy(max_sh, max_all)
    mv = jnp.full((16,), NEG, jnp.float32)
    for q in range(NTEC):
        mv = jnp.maximum(mv, max_all[q])
    return _splat_max(mv)


def _sc_denom_and_norm(ew, dlb, svq, stg, s_sh, zeros1_h, msplat):
    s = lax.axis_index("s")

    @pl.when(s == 0)
    def _():
        pltpu.sync_copy(zeros1_h, s_sh)

    plsc.subcore_barrier()

    def epass(g, _):
        ew[pl.ds(g * 16, 16)] = jnp.exp(ew[pl.ds(g * 16, 16)] - msplat)
        return 0

    lax.fori_loop(0, ETP // 16, epass, 0)

    def sadd(k, _):
        _stage_idx(stg, dlb, k * 128, 128)
        pltpu.sync_copy(ew.at[pl.ds(k * 128, 128)], s_sh.at[stg.at[0]],
                        add=True)
        return 0

    lax.fori_loop(0, ETP // 128, sadd, 0)
    plsc.subcore_barrier()

    for qt in range(4):
        off = qt * QTR

        def sget(k, _):
            _stage_idx(stg, dlb, off + k * 128, 128)
            pltpu.sync_copy(s_sh.at[stg.at[0]], svq.at[pl.ds(k * 128, 128)])
            return 0

        lax.fori_loop(0, QTR // 128, sget, 0)

        def wpass(g, _):
            e = ew[pl.ds(off + g * 16, 16)]
            sv = svq[pl.ds(g * 16, 16)]
            ew[pl.ds(off + g * 16, 16)] = e / (sv + 1e-16)
            return 0

        lax.fori_loop(0, QTR // 16, wpass, 0)


def _sc_aggregate(srcf, dlb, ew, rowsA, rowsB, stgA, stgB, stg2, xs2_h,
                  zerosc_h, h_sh, h_out, semA, semB, base, rg):
    s = lax.axis_index("s")
    ngr = ETP // rg
    bufs = ((rowsA, stgA, semA), (rowsB, stgB, semB))

    for colhalf in range(2):
        @pl.when(s == 0)
        def _():
            pltpu.sync_copy(zerosc_h, h_sh)

        plsc.subcore_barrier()
        _stage_idx(stgA, srcf, 0, rg, add=colhalf * N)
        pltpu.async_copy(xs2_h.at[stgA.at[0]], rowsA, semA)

        def agg2(t, _):
            for par in range(2):
                rows, stg_g, sem_g = bufs[par]
                nrows, nstg, nsem = bufs[1 - par]
                g = 2 * t + par
                gn = lax.rem(g + 1, ngr)
                _stage_idx(nstg, srcf, gn * rg, rg, add=colhalf * N)
                pltpu.async_copy(xs2_h.at[nstg.at[0]], nrows, nsem)
                pltpu.make_async_copy(
                    xs2_h.at[stg_g.at[0]], rows, sem_g).wait()
                wgs = [ew[pl.ds(g * rg + q * 16, 16)]
                       for q in range(rg // 16)]
                for j in range(rg):
                    wj = _vperm(wgs[j // 16],
                                jnp.full((16,), j % 16, jnp.int32))
                    for ch in range(HH // 16):
                        rows[j, pl.ds(ch * 16, 16)] = (
                            rows[j, pl.ds(ch * 16, 16)] * wj)
                _stage_idx(stg2, dlb, g * rg, rg)
                pltpu.sync_copy(rows, h_sh.at[stg2.at[0]], add=True)
            return 0

        lax.fori_loop(0, ngr // 2, agg2, 0)
        pltpu.make_async_copy(xs2_h.at[stgA.at[0]], rowsA, semA).wait()
        plsc.subcore_barrier()

        @pl.when(s == 0)
        def _():
            pltpu.sync_copy(h_sh.at[pl.ds(0, NHALF)],
                            h_out.at[pl.ds(colhalf * N + base, NHALF)])

        plsc.subcore_barrier()


def _sc_gat_body(asrc_h, adst_h, src_h, dst_h, xs2_h, zerosc_h, zeros1_h,
                 h_out,
                 srcf, dlb, ew, dsub, asgq, adg, rowsA, rowsB, maxv,
                 max_all, stg, stgA, stgB, stg2, h_sh, s_sh, max_sh,
                 a1_sh, a2_sh, semA, semB):
    c = lax.axis_index("c")
    s = lax.axis_index("s")
    base = c * NHALF
    e0 = s * ETP

    @pl.when(s == 1)
    def _():
        pltpu.sync_copy(asrc_h, a1_sh)

    @pl.when(s == 2)
    def _():
        pltpu.sync_copy(adst_h, a2_sh)

    pltpu.sync_copy(src_h.at[pl.ds(e0, ETP)], srcf)
    plsc.subcore_barrier()

    lm = jnp.full((16,), NEG, jnp.float32)
    for qt in range(4):
        off = qt * QTR
        pltpu.sync_copy(dst_h.at[pl.ds(e0 + off, QTR)], dsub)

        def dpass(g, _):
            dv = dsub[pl.ds(g * 16, 16)]
            dl = dv - base
            m = (dl >= 0) & (dl < NHALF)
            dlb[pl.ds(off + g * 16, 16)] = jnp.where(m, dl, DUMP)
            return 0

        lax.fori_loop(0, QTR // 16, dpass, 0)

        def gpass_d(k, _):
            _stage_idx_clamped(stg, dsub, k * 128, 128, N - 1)
            pltpu.sync_copy(a2_sh.at[stg.at[0]], adg.at[pl.ds(k * 128, 128)])
            return 0

        lax.fori_loop(0, QTR // 128, gpass_d, 0)

        def gpass_s(k, _):
            _stage_idx(stg, srcf, off + k * 128, 128)
            pltpu.sync_copy(a1_sh.at[stg.at[0]], asgq.at[pl.ds(k * 128, 128)])
            return 0

        lax.fori_loop(0, QTR // 128, gpass_s, 0)

        def apass(g, lmc):
            av = asgq[pl.ds(g * 16, 16)]
            al = _lrelu(av + adg[pl.ds(g * 16, 16)])
            m = dlb[pl.ds(off + g * 16, 16)] < NHALF
            al = jnp.where(m, al, NEG)
            ew[pl.ds(off + g * 16, 16)] = al
            return jnp.maximum(lmc, al)

        lm = lax.fori_loop(0, QTR // 16, apass, lm)

    msplat = _sc_max_combine(lm, maxv, max_all, max_sh)
    _sc_denom_and_norm(ew, dlb, adg, stg, s_sh, zeros1_h, msplat)
    _sc_aggregate(srcf, dlb, ew, rowsA, rowsB, stgA, stgB, stg2, xs2_h,
                  zerosc_h, h_sh, h_out, semA, semB, base, RGA)


def _sc_gate_body(ar_h, g12_h, g22_h, wrow_h, attl_h, ea_h, src_h, dst_h,
                  zerosc_h, zeros1_h, h_out,
                  srcf, dlb, ew, dsub, svq, rowsbA, rowsbB, rowsA, rowsB,
                  wrow_v, attl_v, maxv, max_all, stg, stgA, stgB, stg2,
                  h_sh, s_sh, max_sh, a1_sh, semA, semB):
    c = lax.axis_index("c")
    s = lax.axis_index("s")
    base = c * NHALF
    e0 = s * ETP
    iota16 = lax.iota(jnp.int32, 16)

    @pl.when(s == 1)
    def _():
        pltpu.sync_copy(ar_h, a1_sh)

    pltpu.sync_copy(src_h.at[pl.ds(e0, ETP)], srcf)
    pltpu.sync_copy(wrow_h, wrow_v)
    pltpu.sync_copy(attl_h, attl_v)
    plsc.subcore_barrier()

    for qt in range(4):
        off = qt * QTR
        pltpu.sync_copy(dst_h.at[pl.ds(e0 + off, QTR)], dsub)

        def gpass(k, _):
            _stage_idx_clamped(stg, dsub, k * 128, 128, N - 1)
            pltpu.sync_copy(a1_sh.at[stg.at[0]],
                            svq.at[pl.ds(k * 128, 128)])
            return 0

        lax.fori_loop(0, QTR // 128, gpass, 0)

        def dpass(g, _):
            dv = dsub[pl.ds(g * 16, 16)]
            dl = dv - base
            m = (dl >= 0) & (dl < NHALF)
            dlb[pl.ds(off + g * 16, 16)] = jnp.where(m, dl, DUMP)
            arv = svq[pl.ds(g * 16, 16)]
            ew[pl.ds(off + g * 16, 16)] = jnp.where(m, arv, NEG)
            return 0

        lax.fori_loop(0, QTR // 16, dpass, 0)

    wr = [wrow_v[pl.ds(ch * 16, 16)] for ch in range(H // 16)]
    at = [attl_v[pl.ds(ch * 16, 16)] for ch in range(H // 16)]
    bbufs = ((rowsbA, stgA, semA), (rowsbB, stgB, semB))
    nq = QTR // RGB

    lm = jnp.full((16,), NEG, jnp.float32)
    for qt in range(4):
        off = qt * QTR
        pltpu.sync_copy(ea_h.at[pl.ds(e0 + off, QTR)], svq)
        _stage_idx(stg2, srcf, off, RGB)
        pltpu.async_copy(g12_h.at[stg2.at[0]], rowsbA, semA)

        def bsub2(t, lmc):
            for par in range(2):
                rowsb, stgb, semb = bbufs[par]
                nrows, nstg, nsem = bbufs[1 - par]
                g = 2 * t + par
                gn = lax.rem(g + 1, nq)
                if par == 0:
                    _stage_idx(nstg, srcf, off + gn * RGB, RGB)
                    pltpu.async_copy(g12_h.at[nstg.at[0]], nrows, nsem)
                    pltpu.make_async_copy(
                        g12_h.at[stg2.at[0]], rowsb, semb).wait()
                else:
                    _stage_idx(stg2, srcf, off + gn * RGB, RGB)
                    pltpu.async_copy(g12_h.at[stg2.at[0]], nrows, nsem)
                    pltpu.make_async_copy(
                        g12_h.at[stgb.at[0]], rowsb, semb).wait()
                eav = svq[pl.ds(g * 16, 16)]
                esc = jnp.zeros((16,), jnp.float32)
                for j in range(16):
                    ea_j = _vperm(eav, jnp.full((16,), j, jnp.int32))
                    acc = jnp.zeros((16,), jnp.float32)
                    for ch in range(H // 16):
                        y = (rowsb[j, pl.ds(ch * 16, 16)] + ea_j * wr[ch])
                        y = jnp.where(y >= 0, y, 0.01 * y)
                        acc = acc + y * at[ch]
                    tl = _splat_sum(acc)
                    esc = jnp.where(iota16 == j, tl, esc)
                gbase = off + g * 16
                arv = ew[pl.ds(gbase, 16)]
                al = _lrelu(esc + arv)
                ew[pl.ds(gbase, 16)] = al
                lmc = jnp.maximum(lmc, al)
            return lmc

        lm = lax.fori_loop(0, nq // 2, bsub2, lm)
        pltpu.make_async_copy(g12_h.at[stg2.at[0]], rowsbA, semA).wait()

    msplat = _sc_max_combine(lm, maxv, max_all, max_sh)
    _sc_denom_and_norm(ew, dlb, svq, stg, s_sh, zeros1_h, msplat)
    _sc_aggregate(srcf, dlb, ew, rowsA, rowsB, stgA, stgB, stg2, g22_h,
                  zerosc_h, h_sh, h_out, semA, semB, base, RGB)


@functools.cache
def _sc_mesh():
    return plsc.VectorSubcoreMesh(core_axis_name="c", subcore_axis_name="s")


_F32 = jnp.float32
_I32 = jnp.int32


def _sc_gat_call(asrc, adst, src, dst, xs2, zerosc, zeros1):
    return pl.kernel(
        _sc_gat_body,
        out_type=jax.ShapeDtypeStruct((2 * N, HH), _F32),
        mesh=_sc_mesh(),
        scratch_types=[
            pltpu.VMEM((ETP,), _I32),
            pltpu.VMEM((ETP,), _I32),
            pltpu.VMEM((ETP,), _F32),
            pltpu.VMEM((QTR,), _I32),
            pltpu.VMEM((QTR,), _F32),
            pltpu.VMEM((QTR,), _F32),
            pltpu.VMEM((RGA, HH), _F32),
            pltpu.VMEM((RGA, HH), _F32),
            pltpu.VMEM((16,), _F32),
            pltpu.VMEM((16, 16), _F32),
            pltpu.VMEM((1, 128), _I32),
            pltpu.VMEM((1, RGA), _I32),
            pltpu.VMEM((1, RGA), _I32),
            pltpu.VMEM((1, RGA), _I32),
            pltpu.VMEM_SHARED((SROWS, HH), _F32),
            pltpu.VMEM_SHARED((SROWS,), _F32),
            pltpu.VMEM_SHARED((16, 16), _F32),
            pltpu.VMEM_SHARED((N,), _F32),
            pltpu.VMEM_SHARED((N,), _F32),
            pltpu.SemaphoreType.DMA,
            pltpu.SemaphoreType.DMA,
        ],
    )(asrc, adst, src, dst, xs2, zerosc, zeros1)


def _sc_gate_call(ar, g1, g22, wrow, attl, ea, src, dst, zerosc, zeros1):
    return pl.kernel(
        _sc_gate_body,
        out_type=jax.ShapeDtypeStruct((2 * N, HH), _F32),
        mesh=_sc_mesh(),
        scratch_types=[
            pltpu.VMEM((ETP,), _I32),
            pltpu.VMEM((ETP,), _I32),
            pltpu.VMEM((ETP,), _F32),
            pltpu.VMEM((QTR,), _I32),
            pltpu.VMEM((QTR,), _F32),
            pltpu.VMEM((RGB, H), _F32),
            pltpu.VMEM((RGB, H), _F32),
            pltpu.VMEM((RGB, HH), _F32),
            pltpu.VMEM((RGB, HH), _F32),
            pltpu.VMEM((H,), _F32),
            pltpu.VMEM((H,), _F32),
            pltpu.VMEM((16,), _F32),
            pltpu.VMEM((16, 16), _F32),
            pltpu.VMEM((1, 128), _I32),
            pltpu.VMEM((1, RGB), _I32),
            pltpu.VMEM((1, RGB), _I32),
            pltpu.VMEM((1, RGB), _I32),
            pltpu.VMEM_SHARED((SROWS, HH), _F32),
            pltpu.VMEM_SHARED((SROWS,), _F32),
            pltpu.VMEM_SHARED((16, 16), _F32),
            pltpu.VMEM_SHARED((N,), _F32),
            pltpu.SemaphoreType.DMA,
            pltpu.SemaphoreType.DMA,
        ],
    )(ar, g1, g22, wrow, attl, ea, src, dst, zerosc, zeros1)


_ROWS = 1000
_GRID = N // _ROWS


def _rowspec(width):
    if width is None:
        return pl.BlockSpec((_ROWS, 1), lambda i: (i, 0))
    return pl.BlockSpec((_ROWS, width), lambda i: (i, 0))


def _fullspec(shape):
    nd = len(shape)
    return pl.BlockSpec(shape, lambda i, _n=nd: (0,) * _n)


def _prep_call(x_in, p):
    return pl.pallas_call(
        _prep_body,
        grid=(_GRID,),
        in_specs=[
            _rowspec(3),
            _fullspec((3, H)), _fullspec((H,)),
            _fullspec((H, H)), _fullspec((H, H)), _fullspec((H,)),
        ],
        out_specs=[_rowspec(H), _rowspec(H), _rowspec(H), _rowspec(None)],
        out_shape=[
            jax.ShapeDtypeStruct((N, H), _F32),
            jax.ShapeDtypeStruct((N, H), _F32),
            jax.ShapeDtypeStruct((N, H), _F32),
            jax.ShapeDtypeStruct((N, 1), _F32),
        ],
    )(x_in, p['lin1_w'], p['lin1_b'], p['gate_lin1_w'][:H],
      p['gate_lin2_w'], p['gate_att_r'])


def _post_call(h_raw, x_old, bias, gru, next_w, next_as, next_ad):
    wi, wh, bi, bh = gru
    return pl.pallas_call(
        _post_body,
        grid=(_GRID,),
        in_specs=[
            _rowspec(H), _rowspec(H),
            _fullspec((H,)),
            _fullspec((H, 3 * H)), _fullspec((H, 3 * H)),
            _fullspec((3 * H,)), _fullspec((3 * H,)),
            _fullspec((H, H)), _fullspec((H,)), _fullspec((H,)),
        ],
        out_specs=[_rowspec(H), _rowspec(H), _rowspec(None),
                   _rowspec(None)],
        out_shape=[
            jax.ShapeDtypeStruct((N, H), _F32),
            jax.ShapeDtypeStruct((N, H), _F32),
            jax.ShapeDtypeStruct((N, 1), _F32),
            jax.ShapeDtypeStruct((N, 1), _F32),
        ],
    )(h_raw, x_old, bias, wi.T, wh.T, bi, bh, next_w, next_as, next_ad)


def _mol_call(x, xs, a_src, batch, mc, gru, l2w, l2b):
    wi, wh, bi, bh = gru
    return pl.pallas_call(
        _mol_body,
        in_specs=[
            pl.BlockSpec((N, H), lambda: (0, 0)),
            pl.BlockSpec((N, H), lambda: (0, 0)),
            pl.BlockSpec((N,), lambda: (0,)),
            pl.BlockSpec((N,), lambda: (0,)),
            pl.BlockSpec((H, H), lambda: (0, 0)),
            pl.BlockSpec((H,), lambda: (0,)),
            pl.BlockSpec((H,), lambda: (0,)),
            pl.BlockSpec((H, 3 * H), lambda: (0, 0)),
            pl.BlockSpec((H, 3 * H), lambda: (0, 0)),
            pl.BlockSpec((3 * H,), lambda: (0,)),
            pl.BlockSpec((3 * H,), lambda: (0,)),
            pl.BlockSpec((H, OUT_CH), lambda: (0, 0)),
            pl.BlockSpec((OUT_CH,), lambda: (0,)),
        ],
        out_specs=pl.BlockSpec((G, OUT_CH), lambda: (0, 0)),
        out_shape=jax.ShapeDtypeStruct((G, OUT_CH), _F32),
    )(x, xs, a_src, batch, mc['lin_w'], mc['att_dst'], mc['bias'],
      wi.T, wh.T, bi, bh, l2w, l2b)


def _colsplit(m):
    return m.reshape(N, 2, HH).transpose(1, 0, 2).reshape(2 * N, HH)


def _colmerge(m2):
    return m2.reshape(2, N, HH).transpose(1, 0, 2).reshape(N, H)


def kernel(x, edge_index, edge_attr, batch, params):
    p = params
    pad = EP - E
    src = jnp.concatenate([edge_index[0], jnp.zeros((pad,), _I32)])
    dst = jnp.concatenate([edge_index[1],
                           jnp.full((pad,), jnp.int32(1 << 20))])
    ea = jnp.concatenate([edge_attr[:, 0], jnp.zeros((pad,), _F32)])
    zerosc = jnp.zeros((SROWS, HH), _F32)
    zeros1 = jnp.zeros((SROWS,), _F32)

    x1, g1, g2, ar = _prep_call(x, p)
    ar = ar[:, 0]

    h2 = _sc_gate_call(ar, g1, _colsplit(g2), p['gate_lin1_w'][H],
                       p['gate_att_l'], ea, src, dst, zerosc, zeros1)
    conv0 = p['atom_convs'][0]
    xcur, xs, a_s, a_d = _post_call(
        _colmerge(h2), x1, p['gate_bias'], p['gru0'],
        conv0['lin_w'], conv0['att_src'], conv0['att_dst'])

    for li in range(3):
        conv = p['atom_convs'][li]
        gru = p['atom_grus'][li]
        h2 = _sc_gat_call(a_s[:, 0], a_d[:, 0], src, dst, _colsplit(xs),
                          zerosc, zeros1)
        if li < 2:
            nxt = p['atom_convs'][li + 1]
            nw, nas, nad = nxt['lin_w'], nxt['att_src'], nxt['att_dst']
        else:
            mc = p['mol_conv']
            nw, nas, nad = mc['lin_w'], mc['att_src'], mc['att_dst']
        xcur, xs, a_s, a_d = _post_call(
            _colmerge(h2), xcur, conv['bias'], gru, nw, nas, nad)

    return _mol_call(xcur, xs, a_s[:, 0], batch, p['mol_conv'],
                     p['mol_gru'], p['lin2_w'], p['lin2_b'])
